# baseline (speedup 1.0000x reference)
import functools

import jax
import jax.numpy as jnp
from jax import lax
from jax.experimental import pallas as pl
from jax.experimental.pallas import tpu as pltpu
from jax.experimental.pallas import tpu_sc as plsc

E = 8
D = 2048
H = 4096
O = 2048
N = 8192
RH = 64

RB = 1024
T_R = N // RB

TB = 256
T_T = N // TB
G = T_T + E
KH = 8
HB = H // KH


def _router_a_kernel(x_ref, rw1_ref, rb1_ref, rw2_ref, rb2_ref,
                     oh_ref, cnt_ref):
    k = pl.program_id(0)
    hh = jnp.dot(x_ref[...], rw1_ref[...],
                 preferred_element_type=jnp.float32) + rb1_ref[...]
    hh = jnp.maximum(hh, 0.0)
    logits = jnp.dot(hh, rw2_ref[...],
                     preferred_element_type=jnp.float32) + rb2_ref[...]
    m = jnp.max(logits, axis=1, keepdims=True)
    jio = lax.broadcasted_iota(jnp.int32, (RB, E), 1).astype(jnp.float32)
    b_col = jnp.min(jnp.where(logits == m, jio, float(E)), axis=1,
                    keepdims=True)
    onehot = (jio == b_col).astype(jnp.float32)
    oh_ref[...] = onehot
    cnt_ref[pl.ds(k, 1), :] = jnp.sum(onehot, axis=0, keepdims=True)


def _router_b_kernel(oh_ref, cnt_ref, pos_ref, counts_ref):
    k = pl.program_id(0)
    onehot = oh_ref[...]
    cnts = cnt_ref[...]
    kio = lax.broadcasted_iota(jnp.int32, (T_R, E), 0)
    prior = jnp.sum(jnp.where(kio < k, cnts, 0.0), axis=0, keepdims=True)
    totals = jnp.sum(cnts, axis=0, keepdims=True)
    jio = lax.broadcasted_iota(jnp.int32, (RB, E), 1).astype(jnp.float32)
    b_col = jnp.sum(onehot * jio, axis=1, keepdims=True)
    off_b = jnp.sum(jnp.where(jio < b_col, totals, 0.0), axis=1,
                    keepdims=True)
    prior_b = jnp.sum(onehot * prior, axis=1, keepdims=True)
    tril = (lax.broadcasted_iota(jnp.int32, (RB, RB), 0)
            > lax.broadcasted_iota(jnp.int32, (RB, RB), 1))
    rk = jnp.dot(tril.astype(jnp.float32), onehot,
                 preferred_element_type=jnp.float32)
    rank_b = jnp.sum(rk * onehot, axis=1, keepdims=True)
    pos_ref[...] = (off_b + prior_b + rank_b).astype(jnp.int32)
    counts_ref[...] = totals.astype(jnp.int32)


def _router(x, rw1, rb1, rw2, rb2):
    oh, cnt = pl.pallas_call(
        _router_a_kernel,
        grid=(T_R,),
        in_specs=[
            pl.BlockSpec((RB, D), lambda k: (k, 0)),
            pl.BlockSpec((D, RH), lambda k: (0, 0)),
            pl.BlockSpec((1, RH), lambda k: (0, 0)),
            pl.BlockSpec((RH, E), lambda k: (0, 0)),
            pl.BlockSpec((1, E), lambda k: (0, 0)),
        ],
        out_specs=[
            pl.BlockSpec((RB, E), lambda k: (k, 0)),
            pl.BlockSpec((T_R, E), lambda k: (0, 0)),
        ],
        out_shape=[
            jax.ShapeDtypeStruct((N, E), jnp.float32),
            jax.ShapeDtypeStruct((T_R, E), jnp.float32),
        ],
    )(x, rw1, rb1.reshape(1, RH), rw2, rb2.reshape(1, E))
    return pl.pallas_call(
        _router_b_kernel,
        grid=(T_R,),
        in_specs=[
            pl.BlockSpec((RB, E), lambda k: (k, 0)),
            pl.BlockSpec((T_R, E), lambda k: (0, 0)),
        ],
        out_specs=[
            pl.BlockSpec((RB, 1), lambda k: (k, 0)),
            pl.BlockSpec((1, E), lambda k: (0, 0)),
        ],
        out_shape=[
            jax.ShapeDtypeStruct((N, 1), jnp.int32),
            jax.ShapeDtypeStruct((1, E), jnp.int32),
        ],
    )(oh, cnt)


_SC_CH = 32


def _sc_scatter(x, pos):
    info = plsc.get_sparse_core_info()
    nw = info.num_cores * info.num_subcores
    per_w = N // nw
    nchunk = per_w // _SC_CH
    mesh = plsc.VectorSubcoreMesh(core_axis_name="c", subcore_axis_name="s")

    @functools.partial(
        pl.kernel,
        out_type=jax.ShapeDtypeStruct((N, D), jnp.float32),
        mesh=mesh,
        scratch_types=[
            pltpu.VMEM((_SC_CH,), jnp.int32),
            pltpu.VMEM((_SC_CH, D), jnp.float32),
            pltpu.SemaphoreType.DMA,
        ],
    )
    def scatter_rows(x_hbm, pos_hbm, out_hbm, idx_v, rows_v, sem):
        wid = lax.axis_index("s") * info.num_cores + lax.axis_index("c")
        base = wid * per_w

        def body(c, carry):
            b0 = base + c * _SC_CH
            pltpu.sync_copy(pos_hbm.at[pl.ds(b0, _SC_CH)], idx_v)
            pltpu.sync_copy(x_hbm.at[pl.ds(b0, _SC_CH)], rows_v)
            pltpu.async_copy(rows_v, out_hbm.at[idx_v], sem).wait()
            return carry

        lax.fori_loop(0, nchunk, body, 0)

    return scatter_rows(x, pos)


def _mlp_kernel(r_ref, e_ref, s_ref, t_ref,
                xp_ref, w1_ref, w2_ref, b1_ref, b2_ref, out_ref):
    g = pl.program_id(0)
    kh = pl.program_id(1)
    e = e_ref[g]

    xb = xp_ref[...].astype(jnp.bfloat16)
    w1 = w1_ref[0].astype(jnp.bfloat16)
    b1 = b1_ref[pl.ds(e, 1), pl.ds(kh * HB, HB)]
    h = jnp.dot(xb, w1, preferred_element_type=jnp.float32) + b1
    h = jnp.maximum(h, 0.0).astype(jnp.bfloat16)
    w2 = w2_ref[0].astype(jnp.bfloat16)
    contrib = jnp.dot(h, w2, preferred_element_type=jnp.float32)

    scale = jnp.where(kh == 0, 1.0, 0.0).astype(jnp.float32)
    contrib = contrib + b2_ref[pl.ds(e, 1), :] * scale

    row = (r_ref[g] * TB
           + lax.broadcasted_iota(jnp.int32, (TB, 1), 0))
    mask = (row >= s_ref[g]) & (row < t_ref[g])
    contrib = jnp.where(mask, contrib, 0.0)

    gm1 = jnp.maximum(g - 1, 0)
    init = jnp.logical_and(jnp.logical_or(g == 0, r_ref[g] != r_ref[gm1]),
                           kh == 0)

    @pl.when(init)
    def _():
        out_ref[...] = contrib

    @pl.when(jnp.logical_not(init))
    def _():
        out_ref[...] += contrib


def _grouped_mlp(seg_r, seg_e, seg_s, seg_t, xp, bw1, bb1, bw2, bb2):
    grid_spec = pltpu.PrefetchScalarGridSpec(
        num_scalar_prefetch=4,
        grid=(G, KH),
        in_specs=[
            pl.BlockSpec((TB, D), lambda g, kh, r, e, s, t: (r[g], 0)),
            pl.BlockSpec((1, D, HB), lambda g, kh, r, e, s, t: (e[g], 0, kh)),
            pl.BlockSpec((1, HB, O), lambda g, kh, r, e, s, t: (e[g], kh, 0)),
            pl.BlockSpec((E, H), lambda g, kh, r, e, s, t: (0, 0)),
            pl.BlockSpec((E, O), lambda g, kh, r, e, s, t: (0, 0)),
        ],
        out_specs=pl.BlockSpec((TB, O), lambda g, kh, r, e, s, t: (r[g], 0)),
    )
    return pl.pallas_call(
        _mlp_kernel,
        grid_spec=grid_spec,
        out_shape=jax.ShapeDtypeStruct((N, O), jnp.float32),
    )(seg_r, seg_e, seg_s, seg_t, xp, bw1, bw2, bb1, bb2)


def kernel(x, rw1, rb1, rw2, rb2, bw1, bb1, bw2, bb2):
    pos2d, counts2d = _router(x, rw1, rb1, rw2, rb2)
    xp = _sc_scatter(x, pos2d.reshape(N))

    counts = counts2d.reshape(E)
    off = jnp.concatenate([jnp.zeros((1,), jnp.int32),
                           jnp.cumsum(counts).astype(jnp.int32)])
    bounds = jnp.sort(jnp.concatenate(
        [jnp.arange(T_T, dtype=jnp.int32) * TB, off[:E]]))
    seg_s = bounds
    seg_t = jnp.concatenate([bounds[1:], jnp.array([N], jnp.int32)])
    seg_r = jnp.minimum(bounds // TB, T_T - 1)
    seg_e = jnp.clip(
        jnp.searchsorted(off, bounds, side="right").astype(jnp.int32) - 1,
        0, E - 1)

    return _grouped_mlp(seg_r, seg_e, seg_s, seg_t, xp, bw1, bb1, bw2, bb2)


# speedup vs baseline: 1.4782x; 1.4782x over previous
import functools

import jax
import jax.numpy as jnp
from jax import lax
from jax.experimental import pallas as pl
from jax.experimental.pallas import tpu as pltpu
from jax.experimental.pallas import tpu_sc as plsc

E = 8
D = 2048
H = 4096
O = 2048
N = 8192
RH = 64

RB = 1024
T_R = N // RB

TB = 512
T_T = N // TB
G = T_T + E
KH = 4
HB = H // KH


def _router_a_kernel(x_ref, rw1_ref, rb1_ref, rw2_ref, rb2_ref,
                     oh_ref, cnt_ref):
    k = pl.program_id(0)
    hh = jnp.dot(x_ref[...], rw1_ref[...],
                 preferred_element_type=jnp.float32) + rb1_ref[...]
    hh = jnp.maximum(hh, 0.0)
    logits = jnp.dot(hh, rw2_ref[...],
                     preferred_element_type=jnp.float32) + rb2_ref[...]
    m = jnp.max(logits, axis=1, keepdims=True)
    jio = lax.broadcasted_iota(jnp.int32, (RB, E), 1).astype(jnp.float32)
    b_col = jnp.min(jnp.where(logits == m, jio, float(E)), axis=1,
                    keepdims=True)
    onehot = (jio == b_col).astype(jnp.float32)
    oh_ref[...] = onehot
    cnt_ref[pl.ds(k, 1), :] = jnp.sum(onehot, axis=0, keepdims=True)


def _router_b_kernel(oh_ref, cnt_ref, pos_ref, counts_ref):
    k = pl.program_id(0)
    onehot = oh_ref[...]
    cnts = cnt_ref[...]
    kio = lax.broadcasted_iota(jnp.int32, (T_R, E), 0)
    prior = jnp.sum(jnp.where(kio < k, cnts, 0.0), axis=0, keepdims=True)
    totals = jnp.sum(cnts, axis=0, keepdims=True)
    jio = lax.broadcasted_iota(jnp.int32, (RB, E), 1).astype(jnp.float32)
    b_col = jnp.sum(onehot * jio, axis=1, keepdims=True)
    off_b = jnp.sum(jnp.where(jio < b_col, totals, 0.0), axis=1,
                    keepdims=True)
    prior_b = jnp.sum(onehot * prior, axis=1, keepdims=True)
    tril = (lax.broadcasted_iota(jnp.int32, (RB, RB), 0)
            > lax.broadcasted_iota(jnp.int32, (RB, RB), 1))
    rk = jnp.dot(tril.astype(jnp.float32), onehot,
                 preferred_element_type=jnp.float32)
    rank_b = jnp.sum(rk * onehot, axis=1, keepdims=True)
    pos_ref[...] = (off_b + prior_b + rank_b).astype(jnp.int32)
    counts_ref[...] = totals.astype(jnp.int32)


def _router(x, rw1, rb1, rw2, rb2):
    oh, cnt = pl.pallas_call(
        _router_a_kernel,
        grid=(T_R,),
        in_specs=[
            pl.BlockSpec((RB, D), lambda k: (k, 0)),
            pl.BlockSpec((D, RH), lambda k: (0, 0)),
            pl.BlockSpec((1, RH), lambda k: (0, 0)),
            pl.BlockSpec((RH, E), lambda k: (0, 0)),
            pl.BlockSpec((1, E), lambda k: (0, 0)),
        ],
        out_specs=[
            pl.BlockSpec((RB, E), lambda k: (k, 0)),
            pl.BlockSpec((T_R, E), lambda k: (0, 0)),
        ],
        out_shape=[
            jax.ShapeDtypeStruct((N, E), jnp.float32),
            jax.ShapeDtypeStruct((T_R, E), jnp.float32),
        ],
    )(x, rw1, rb1.reshape(1, RH), rw2, rb2.reshape(1, E))
    return pl.pallas_call(
        _router_b_kernel,
        grid=(T_R,),
        in_specs=[
            pl.BlockSpec((RB, E), lambda k: (k, 0)),
            pl.BlockSpec((T_R, E), lambda k: (0, 0)),
        ],
        out_specs=[
            pl.BlockSpec((RB, 1), lambda k: (k, 0)),
            pl.BlockSpec((1, E), lambda k: (0, 0)),
        ],
        out_shape=[
            jax.ShapeDtypeStruct((N, 1), jnp.int32),
            jax.ShapeDtypeStruct((1, E), jnp.int32),
        ],
    )(oh, cnt)


_SC_CH = 32


def _sc_scatter(x, pos):
    info = plsc.get_sparse_core_info()
    nw = info.num_cores * info.num_subcores
    per_w = N // nw
    nchunk = per_w // _SC_CH
    mesh = plsc.VectorSubcoreMesh(core_axis_name="c", subcore_axis_name="s")

    @functools.partial(
        pl.kernel,
        out_type=jax.ShapeDtypeStruct((N, D), jnp.float32),
        mesh=mesh,
        scratch_types=[
            pltpu.VMEM((_SC_CH,), jnp.int32),
            pltpu.VMEM((_SC_CH, D), jnp.float32),
            pltpu.SemaphoreType.DMA,
        ],
    )
    def scatter_rows(x_hbm, pos_hbm, out_hbm, idx_v, rows_v, sem):
        wid = lax.axis_index("s") * info.num_cores + lax.axis_index("c")
        base = wid * per_w

        def body(c, carry):
            b0 = base + c * _SC_CH
            pltpu.sync_copy(pos_hbm.at[pl.ds(b0, _SC_CH)], idx_v)
            pltpu.sync_copy(x_hbm.at[pl.ds(b0, _SC_CH)], rows_v)
            pltpu.async_copy(rows_v, out_hbm.at[idx_v], sem).wait()
            return carry

        lax.fori_loop(0, nchunk, body, 0)

    return scatter_rows(x, pos)


def _mlp_kernel(r_ref, e_ref, s_ref, t_ref,
                xp_ref, w1_ref, w2_ref, b1_ref, b2_ref, out_ref):
    g = pl.program_id(0)
    kh = pl.program_id(1)
    e = e_ref[g]

    xb = xp_ref[...].astype(jnp.bfloat16)
    w1 = w1_ref[0].astype(jnp.bfloat16)
    b1 = b1_ref[pl.ds(e, 1), pl.ds(kh * HB, HB)]
    h = jnp.dot(xb, w1, preferred_element_type=jnp.float32) + b1
    h = jnp.maximum(h, 0.0).astype(jnp.bfloat16)
    w2 = w2_ref[0].astype(jnp.bfloat16)
    contrib = jnp.dot(h, w2, preferred_element_type=jnp.float32)

    scale = jnp.where(kh == 0, 1.0, 0.0).astype(jnp.float32)
    contrib = contrib + b2_ref[pl.ds(e, 1), :] * scale

    row = (r_ref[g] * TB
           + lax.broadcasted_iota(jnp.int32, (TB, 1), 0))
    mask = (row >= s_ref[g]) & (row < t_ref[g])
    contrib = jnp.where(mask, contrib, 0.0)

    gm1 = jnp.maximum(g - 1, 0)
    init = jnp.logical_and(jnp.logical_or(g == 0, r_ref[g] != r_ref[gm1]),
                           kh == 0)

    @pl.when(init)
    def _():
        out_ref[...] = contrib

    @pl.when(jnp.logical_not(init))
    def _():
        out_ref[...] += contrib


def _grouped_mlp(seg_r, seg_e, seg_s, seg_t, xp, bw1, bb1, bw2, bb2):
    grid_spec = pltpu.PrefetchScalarGridSpec(
        num_scalar_prefetch=4,
        grid=(G, KH),
        in_specs=[
            pl.BlockSpec((TB, D), lambda g, kh, r, e, s, t: (r[g], 0)),
            pl.BlockSpec((1, D, HB), lambda g, kh, r, e, s, t: (e[g], 0, kh)),
            pl.BlockSpec((1, HB, O), lambda g, kh, r, e, s, t: (e[g], kh, 0)),
            pl.BlockSpec((E, H), lambda g, kh, r, e, s, t: (0, 0)),
            pl.BlockSpec((E, O), lambda g, kh, r, e, s, t: (0, 0)),
        ],
        out_specs=pl.BlockSpec((TB, O), lambda g, kh, r, e, s, t: (r[g], 0)),
    )
    return pl.pallas_call(
        _mlp_kernel,
        grid_spec=grid_spec,
        out_shape=jax.ShapeDtypeStruct((N, O), jnp.float32),
    )(seg_r, seg_e, seg_s, seg_t, xp, bw1, bw2, bb1, bb2)


def kernel(x, rw1, rb1, rw2, rb2, bw1, bb1, bw2, bb2):
    pos2d, counts2d = _router(x, rw1, rb1, rw2, rb2)
    xp = _sc_scatter(x, pos2d.reshape(N))

    counts = counts2d.reshape(E)
    off = jnp.concatenate([jnp.zeros((1,), jnp.int32),
                           jnp.cumsum(counts).astype(jnp.int32)])
    bounds = jnp.sort(jnp.concatenate(
        [jnp.arange(T_T, dtype=jnp.int32) * TB, off[:E]]))
    seg_s = bounds
    seg_t = jnp.concatenate([bounds[1:], jnp.array([N], jnp.int32)])
    seg_r = jnp.minimum(bounds // TB, T_T - 1)
    seg_e = jnp.clip(
        jnp.searchsorted(off, bounds, side="right").astype(jnp.int32) - 1,
        0, E - 1)
    return _grouped_mlp(seg_r, seg_e, seg_s, seg_t, xp, bw1, bb1, bw2, bb2)
